# baseline (speedup 1.0000x reference)
import jax
import jax.numpy as jnp
from jax import lax
from jax.experimental import pallas as pl
from jax.experimental.pallas import tpu as pltpu
from jax.experimental.pallas import tpu_sc as plsc

N = 10000
E = 320000
D = 128
H = 256
HE = 320
OUT = 64

NP_ = 10240
EP = 327680
ER = EP // 128
ZR = NP_ // 16
NBLK = 512
NB = NP_ // NBLK


def _sc_agg(h2, src_both, dstp, zrows, esplit):
    nrows = 80 if esplit else 160
    QR = 40
    NQ = nrows // QR
    mesh = plsc.VectorSubcoreMesh(core_axis_name="c", subcore_axis_name="s")
    scratch = [
        pltpu.VMEM_SHARED((NP_, 128), jnp.float32),
        pltpu.VMEM((QR, 128), jnp.int32),
        pltpu.VMEM((QR, 128), jnp.int32),
        pltpu.VMEM((2, 128, 128), jnp.float32),
        pltpu.SemaphoreType.DMA,
    ]

    def body(h_ref, srcb_ref, dst_ref, zr_ref, out_ref,
             acc, isrc, idst, rows, gsem):
        c = lax.axis_index("c")
        s = lax.axis_index("s")

        pltpu.sync_copy(zr_ref, rows.at[0])
        for j in range(ZR // 128):
            pltpu.sync_copy(rows.at[0], acc.at[pl.ds(s * ZR + j * 128, 128)])
        plsc.subcore_barrier()

        base = (c * 16 + s) * nrows if esplit else s * nrows

        def quarter(q, carry):
            qbase = base + q * QR
            pltpu.sync_copy(srcb_ref.at[c, pl.ds(qbase, QR)], isrc)
            pltpu.sync_copy(dst_ref.at[pl.ds(qbase, QR)], idst)
            pltpu.async_copy(h_ref.at[isrc.at[0]], rows.at[0], gsem)
            pltpu.async_copy(h_ref.at[isrc.at[1]], rows.at[1], gsem)

            def step(r, cc):
                b = r % 2
                pltpu.make_async_copy(
                    h_ref.at[pl.ds(0, 128)], rows.at[b], gsem).wait()
                pltpu.sync_copy(rows.at[b], acc.at[idst.at[r]], add=True)

                @pl.when(r < QR - 2)
                def _():
                    pltpu.async_copy(
                        h_ref.at[isrc.at[r + 2]], rows.at[b], gsem)

                return cc

            lax.fori_loop(0, QR, step, 0)
            return carry

        lax.fori_loop(0, NQ, quarter, 0)
        plsc.subcore_barrier()

        for j in range(ZR // 128):
            pltpu.sync_copy(acc.at[pl.ds(s * ZR + j * 128, 128)], rows.at[0])
            pltpu.sync_copy(rows.at[0],
                            out_ref.at[c, pl.ds(s * ZR + j * 128, 128)])

    fn = pl.kernel(
        body,
        out_type=jax.ShapeDtypeStruct((2, NP_, 128), jnp.float32),
        mesh=mesh, scratch_types=scratch)
    return fn(h2, src_both, dstp, zrows)


def _sc_deg(dstp, zrows, ones):
    nrows = 80
    mesh = plsc.VectorSubcoreMesh(core_axis_name="c", subcore_axis_name="s")
    scratch = [
        pltpu.VMEM_SHARED((NP_, 128), jnp.float32),
        pltpu.VMEM((nrows, 128), jnp.int32),
        pltpu.VMEM((128, 128), jnp.float32),
    ]

    def body(dst_ref, zr_ref, ones_ref, out_ref, degacc, idst, ones_v):
        c = lax.axis_index("c")
        s = lax.axis_index("s")
        pltpu.sync_copy(zr_ref, ones_v)
        for j in range(ZR // 128):
            pltpu.sync_copy(ones_v,
                            degacc.at[pl.ds(s * ZR + j * 128, 128)])
        base = (c * 16 + s) * nrows
        pltpu.sync_copy(dst_ref.at[pl.ds(base, nrows)], idst)
        pltpu.sync_copy(ones_ref, ones_v)
        plsc.subcore_barrier()

        def step(r, cc):
            pltpu.sync_copy(ones_v, degacc.at[idst.at[r]], add=True)
            return cc

        lax.fori_loop(0, nrows, step, 0)
        plsc.subcore_barrier()
        for j in range(ZR // 128):
            pltpu.sync_copy(degacc.at[pl.ds(s * ZR + j * 128, 128)], ones_v)
            pltpu.sync_copy(ones_v,
                            out_ref.at[c, pl.ds(s * ZR + j * 128, 128)])

    fn = pl.kernel(
        body,
        out_type=jax.ShapeDtypeStruct((2, NP_, 128), jnp.float32),
        mesh=mesh, scratch_types=scratch)
    return fn(dstp, zrows, ones)


def _pass1h(h2, Wr, bl, Din, Hout, esplit):
    Dc = Din // 2

    def body(h0_ref, h1_ref, wr_ref, bl_ref, z_ref):
        if esplit:
            z = jnp.dot(h0_ref[...], wr_ref[...],
                        preferred_element_type=jnp.float32) + bl_ref[...]
        else:
            z = (jnp.dot(h0_ref[...], wr_ref[0:Dc, :],
                         preferred_element_type=jnp.float32)
                 + jnp.dot(h1_ref[...], wr_ref[Dc:, :],
                           preferred_element_type=jnp.float32)
                 + bl_ref[...])
        z_ref[...] = z

    if esplit:
        h_specs = [pl.BlockSpec((NBLK, Din), lambda r: (r, 0)),
                   pl.BlockSpec((NBLK, Din), lambda r: (r, 0))]
    else:
        h_specs = [pl.BlockSpec((NBLK, Dc), lambda r: (r, 0)),
                   pl.BlockSpec((NBLK, Dc), lambda r: (r + NB, 0))]

    return pl.pallas_call(
        body,
        grid=(NB,),
        in_specs=h_specs + [
            pl.BlockSpec((Din, Hout), lambda r: (0, 0)),
            pl.BlockSpec((1, Hout), lambda r: (0, 0)),
        ],
        out_specs=pl.BlockSpec((NBLK, Hout), lambda r: (r, 0)),
        out_shape=jax.ShapeDtypeStruct((NP_, Hout), jnp.float32),
    )(h2, h2, Wr, bl.reshape(1, Hout))


def _pass1a(agg2, deg2, z0, Wl, Din, Hout, esplit):
    Dc = Din // 2
    aggr = agg2.reshape(2 * NP_, 128)

    def body(a0_ref, a1_ref, d0_ref, d1_ref, z0_ref, wl_ref,
             z_ref, sums_ref):
        rb = pl.program_id(0)

        @pl.when(rb == 0)
        def _():
            sums_ref[...] = jnp.zeros_like(sums_ref)

        deg = d0_ref[0] + d1_ref[0]
        recip = 1.0 / jnp.maximum(deg, 1.0)
        if esplit:
            a = (a0_ref[...] + a1_ref[...]) * recip
            z = z0_ref[...] + jnp.dot(a, wl_ref[...],
                                      preferred_element_type=jnp.float32)
        else:
            z = (z0_ref[...]
                 + jnp.dot(a0_ref[...] * recip, wl_ref[0:Dc, :],
                           preferred_element_type=jnp.float32)
                 + jnp.dot(a1_ref[...] * recip, wl_ref[Dc:, :],
                           preferred_element_type=jnp.float32))
        rid = rb * NBLK + lax.broadcasted_iota(jnp.int32, (NBLK, 1), 0)
        zm = jnp.where(rid < N, z, 0.0)
        sums_ref[0:1, :] += jnp.sum(zm, axis=0, keepdims=True)
        sums_ref[1:2, :] += jnp.sum(zm * zm, axis=0, keepdims=True)
        z_ref[...] = z

    return pl.pallas_call(
        body,
        grid=(NB,),
        in_specs=[
            pl.BlockSpec((NBLK, 128), lambda r: (r, 0)),
            pl.BlockSpec((NBLK, 128), lambda r: (r + NB, 0)),
            pl.BlockSpec((1, NBLK, 1), lambda r: (0, r, 0)),
            pl.BlockSpec((1, NBLK, 1), lambda r: (1, r, 0)),
            pl.BlockSpec((NBLK, Hout), lambda r: (r, 0)),
            pl.BlockSpec((Din, Hout), lambda r: (0, 0)),
        ],
        out_specs=[
            pl.BlockSpec((NBLK, Hout), lambda r: (r, 0)),
            pl.BlockSpec((2, Hout), lambda r: (0, 0)),
        ],
        out_shape=[
            jax.ShapeDtypeStruct((NP_, Hout), jnp.float32),
            jax.ShapeDtypeStruct((2, Hout), jnp.float32),
        ],
    )(aggr, aggr, deg2, deg2, z0, Wl)


def _pass2_split(z, sums, g, be, Hout):
    Dc = Hout // 2

    def body(z_ref, sums_ref, g_ref, be_ref, out_ref):
        m = sums_ref[0:1, :] * (1.0 / N)
        var = sums_ref[1:2, :] * (1.0 / N) - m * m
        inv = lax.rsqrt(var + 1e-5)
        y = (z_ref[...] - m) * (inv * g_ref[...]) + be_ref[...]
        out_ref[0, :, :] = jnp.maximum(y, 0.0)

    return pl.pallas_call(
        body,
        grid=(2, NB),
        in_specs=[
            pl.BlockSpec((NBLK, Dc), lambda c, r: (r, c)),
            pl.BlockSpec((2, Dc), lambda c, r: (0, c)),
            pl.BlockSpec((1, Dc), lambda c, r: (0, c)),
            pl.BlockSpec((1, Dc), lambda c, r: (0, c)),
        ],
        out_specs=pl.BlockSpec((1, NBLK, Dc), lambda c, r: (c, r, 0)),
        out_shape=jax.ShapeDtypeStruct((2, NP_, Dc), jnp.float32),
    )(z, sums, g.reshape(1, Hout), be.reshape(1, Hout))


def _pass2_head(z, sums, g, be, Lw1, Lb1, Lw2, Lb2):
    nb = 400

    def body(z_ref, sums_ref, g_ref, be_ref, w1_ref, b1_ref, w2_ref, b2_ref,
             out_ref):
        m = sums_ref[0:1, :] * (1.0 / N)
        var = sums_ref[1:2, :] * (1.0 / N) - m * m
        inv = lax.rsqrt(var + 1e-5)
        h = (z_ref[...] - m) * (inv * g_ref[...]) + be_ref[...]
        h = jnp.maximum(h, 0.0)
        t = jnp.dot(h, w1_ref[...], preferred_element_type=jnp.float32)
        t = jnp.maximum(t + b1_ref[...], 0.0)
        out_ref[...] = jnp.dot(
            t, w2_ref[...], preferred_element_type=jnp.float32) + b2_ref[...]

    return pl.pallas_call(
        body,
        grid=(N // nb,),
        in_specs=[
            pl.BlockSpec((nb, HE), lambda r: (r, 0)),
            pl.BlockSpec((2, HE), lambda r: (0, 0)),
            pl.BlockSpec((1, HE), lambda r: (0, 0)),
            pl.BlockSpec((1, HE), lambda r: (0, 0)),
            pl.BlockSpec((HE, H), lambda r: (0, 0)),
            pl.BlockSpec((1, H), lambda r: (0, 0)),
            pl.BlockSpec((H, OUT), lambda r: (0, 0)),
            pl.BlockSpec((1, OUT), lambda r: (0, 0)),
        ],
        out_specs=pl.BlockSpec((nb, OUT), lambda r: (r, 0)),
        out_shape=jax.ShapeDtypeStruct((N, OUT), jnp.float32),
    )(z, sums, g.reshape(1, HE), be.reshape(1, HE), Lw1,
      Lb1.reshape(1, H), Lw2, Lb2.reshape(1, OUT))


def kernel(x, edge_index, W1l, b1l, W1r, W2l, b2l, W2r, W3l, b3l, W3r,
           g1, be1, g2, be2, g3, be3, Lw1, Lb1, Lw2, Lb2):
    src = edge_index[0]
    dst = edge_index[1]
    pad = EP - E
    srcp = jnp.concatenate(
        [src, jnp.zeros((pad,), jnp.int32)]).reshape(ER, 128)
    pad_dst = N + jnp.arange(pad, dtype=jnp.int32) % (NP_ - N)
    dstp = jnp.concatenate([dst, pad_dst]).reshape(ER, 128)
    src_same = jnp.stack([srcp, srcp])
    src_off = jnp.stack([srcp, srcp + NP_])

    zrows = jnp.zeros((128, 128), jnp.float32)
    ones = jnp.ones((128, 128), jnp.float32)

    x2 = jnp.pad(x, ((0, NP_ - N), (0, 0)))

    deg2 = _sc_deg(dstp, zrows, ones)[:, :, :1]

    z0_1 = _pass1h(x2, W1r, b1l, D, H, True)
    agg1 = _sc_agg(x2, src_same, dstp, zrows, True)
    z1, s1 = _pass1a(agg1, deg2, z0_1, W1l, D, H, True)
    h1 = _pass2_split(z1, s1, g1, be1, H).reshape(2 * NP_, H // 2)

    z0_2 = _pass1h(h1, W2r, b2l, H, H, False)
    agg2 = _sc_agg(h1, src_off, dstp, zrows, False)
    z2, s2 = _pass1a(agg2, deg2, z0_2, W2l, H, H, False)
    h2 = _pass2_split(z2, s2, g2, be2, H).reshape(2 * NP_, H // 2)

    z0_3 = _pass1h(h2, W3r, b3l, H, HE, False)
    agg3 = _sc_agg(h2, src_off, dstp, zrows, False)
    z3, s3 = _pass1a(agg3, deg2, z0_3, W3l, H, HE, False)
    return _pass2_head(z3, s3, g3, be3, Lw1, Lb1, Lw2, Lb2)


# speedup vs baseline: 1.1095x; 1.1095x over previous
import jax
import jax.numpy as jnp
from jax import lax
from jax.experimental import pallas as pl
from jax.experimental.pallas import tpu as pltpu
from jax.experimental.pallas import tpu_sc as plsc

N = 10000
E = 320000
D = 128
H = 256
HE = 320
OUT = 64

NP_ = 10240
EP = 327680
ER = EP // 128
ZR = NP_ // 16
NBLK = 512
NB = NP_ // NBLK


def _sc_agg(h2, src_both, dstp, zrows, esplit):
    nrows = 80 if esplit else 160
    QR = 40
    NQ = nrows // QR
    mesh = plsc.VectorSubcoreMesh(core_axis_name="c", subcore_axis_name="s")
    scratch = [
        pltpu.VMEM_SHARED((NP_, 128), jnp.float32),
        pltpu.VMEM((QR, 128), jnp.int32),
        pltpu.VMEM((QR, 128), jnp.int32),
        pltpu.VMEM((2, 128, 128), jnp.float32),
        pltpu.SemaphoreType.DMA,
    ]

    def body(h_ref, srcb_ref, dst_ref, zr_ref, out_ref,
             acc, isrc, idst, rows, gsem):
        c = lax.axis_index("c")
        s = lax.axis_index("s")
        col = pl.multiple_of(c * 128, 128)

        def gather_src(r):
            if esplit:
                return h_ref.at[isrc.at[r]]
            return h_ref.at[isrc.at[r], pl.ds(col, 128)]

        pltpu.sync_copy(zr_ref, rows.at[0])
        for j in range(ZR // 128):
            pltpu.sync_copy(rows.at[0], acc.at[pl.ds(s * ZR + j * 128, 128)])
        plsc.subcore_barrier()

        base = (c * 16 + s) * nrows if esplit else s * nrows

        def quarter(q, carry):
            qbase = base + q * QR
            pltpu.sync_copy(srcb_ref.at[c, pl.ds(qbase, QR)], isrc)
            pltpu.sync_copy(dst_ref.at[pl.ds(qbase, QR)], idst)
            pltpu.async_copy(gather_src(0), rows.at[0], gsem)
            pltpu.async_copy(gather_src(1), rows.at[1], gsem)

            def step(r, cc):
                b = r % 2
                pltpu.make_async_copy(
                    h_ref.at[pl.ds(0, 128), pl.ds(0, 128)]
                    if not esplit else h_ref.at[pl.ds(0, 128)],
                    rows.at[b], gsem).wait()
                pltpu.sync_copy(rows.at[b], acc.at[idst.at[r]], add=True)

                @pl.when(r < QR - 2)
                def _():
                    pltpu.async_copy(gather_src(r + 2), rows.at[b], gsem)

                return cc

            lax.fori_loop(0, QR, step, 0)
            return carry

        lax.fori_loop(0, NQ, quarter, 0)
        plsc.subcore_barrier()

        for j in range(ZR // 128):
            pltpu.sync_copy(acc.at[pl.ds(s * ZR + j * 128, 128)], rows.at[0])
            pltpu.sync_copy(rows.at[0],
                            out_ref.at[c, pl.ds(s * ZR + j * 128, 128)])

    fn = pl.kernel(
        body,
        out_type=jax.ShapeDtypeStruct((2, NP_, 128), jnp.float32),
        mesh=mesh, scratch_types=scratch)
    return fn(h2, src_both, dstp, zrows)


def _sc_deg(dstp, zrows, ones):
    nrows = 80
    mesh = plsc.VectorSubcoreMesh(core_axis_name="c", subcore_axis_name="s")
    scratch = [
        pltpu.VMEM_SHARED((NP_, 128), jnp.float32),
        pltpu.VMEM((nrows, 128), jnp.int32),
        pltpu.VMEM((128, 128), jnp.float32),
    ]

    def body(dst_ref, zr_ref, ones_ref, out_ref, degacc, idst, ones_v):
        c = lax.axis_index("c")
        s = lax.axis_index("s")
        pltpu.sync_copy(zr_ref, ones_v)
        for j in range(ZR // 128):
            pltpu.sync_copy(ones_v,
                            degacc.at[pl.ds(s * ZR + j * 128, 128)])
        base = (c * 16 + s) * nrows
        pltpu.sync_copy(dst_ref.at[pl.ds(base, nrows)], idst)
        pltpu.sync_copy(ones_ref, ones_v)
        plsc.subcore_barrier()

        def step(r, cc):
            pltpu.sync_copy(ones_v, degacc.at[idst.at[r]], add=True)
            return cc

        lax.fori_loop(0, nrows, step, 0)
        plsc.subcore_barrier()
        for j in range(ZR // 128):
            pltpu.sync_copy(degacc.at[pl.ds(s * ZR + j * 128, 128)], ones_v)
            pltpu.sync_copy(ones_v,
                            out_ref.at[c, pl.ds(s * ZR + j * 128, 128)])

    fn = pl.kernel(
        body,
        out_type=jax.ShapeDtypeStruct((2, NP_, 128), jnp.float32),
        mesh=mesh, scratch_types=scratch)
    return fn(dstp, zrows, ones)


def _pass1(agg2, h2, deg2, Wl, bl, Wr, Din, Hout, esplit):
    aggr = agg2.reshape(2 * NP_, 128)

    def body(a0_ref, a1_ref, h_ref, d0_ref, d1_ref, wl_ref, bl_ref,
             wr_ref, z_ref, sums_ref):
        rb = pl.program_id(0)

        @pl.when(rb == 0)
        def _():
            sums_ref[...] = jnp.zeros_like(sums_ref)

        deg = d0_ref[0] + d1_ref[0]
        recip = 1.0 / jnp.maximum(deg, 1.0)
        hw = jnp.dot(h_ref[...], wr_ref[...],
                     preferred_element_type=jnp.float32)
        if esplit:
            a = (a0_ref[...] + a1_ref[...]) * recip
            z = hw + jnp.dot(a, wl_ref[...],
                             preferred_element_type=jnp.float32) + bl_ref[...]
        else:
            z = (hw
                 + jnp.dot(a0_ref[...] * recip, wl_ref[0:128, :],
                           preferred_element_type=jnp.float32)
                 + jnp.dot(a1_ref[...] * recip, wl_ref[128:, :],
                           preferred_element_type=jnp.float32)
                 + bl_ref[...])
        rid = rb * NBLK + lax.broadcasted_iota(jnp.int32, (NBLK, 1), 0)
        zm = jnp.where(rid < N, z, 0.0)
        sums_ref[0:1, :] += jnp.sum(zm, axis=0, keepdims=True)
        sums_ref[1:2, :] += jnp.sum(zm * zm, axis=0, keepdims=True)
        z_ref[...] = z

    return pl.pallas_call(
        body,
        grid=(NB,),
        in_specs=[
            pl.BlockSpec((NBLK, 128), lambda r: (r, 0)),
            pl.BlockSpec((NBLK, 128), lambda r: (r + NB, 0)),
            pl.BlockSpec((NBLK, Din), lambda r: (r, 0)),
            pl.BlockSpec((1, NBLK, 1), lambda r: (0, r, 0)),
            pl.BlockSpec((1, NBLK, 1), lambda r: (1, r, 0)),
            pl.BlockSpec((Din, Hout), lambda r: (0, 0)),
            pl.BlockSpec((1, Hout), lambda r: (0, 0)),
            pl.BlockSpec((Din, Hout), lambda r: (0, 0)),
        ],
        out_specs=[
            pl.BlockSpec((NBLK, Hout), lambda r: (r, 0)),
            pl.BlockSpec((2, Hout), lambda r: (0, 0)),
        ],
        out_shape=[
            jax.ShapeDtypeStruct((NP_, Hout), jnp.float32),
            jax.ShapeDtypeStruct((2, Hout), jnp.float32),
        ],
    )(aggr, aggr, h2, deg2, deg2, Wl, bl.reshape(1, Hout), Wr)


def _pass2(z, sums, g, be, Hout):
    def body(z_ref, sums_ref, g_ref, be_ref, out_ref):
        m = sums_ref[0:1, :] * (1.0 / N)
        var = sums_ref[1:2, :] * (1.0 / N) - m * m
        inv = lax.rsqrt(var + 1e-5)
        y = (z_ref[...] - m) * (inv * g_ref[...]) + be_ref[...]
        out_ref[...] = jnp.maximum(y, 0.0)

    return pl.pallas_call(
        body,
        grid=(NB,),
        in_specs=[
            pl.BlockSpec((NBLK, Hout), lambda r: (r, 0)),
            pl.BlockSpec((2, Hout), lambda r: (0, 0)),
            pl.BlockSpec((1, Hout), lambda r: (0, 0)),
            pl.BlockSpec((1, Hout), lambda r: (0, 0)),
        ],
        out_specs=pl.BlockSpec((NBLK, Hout), lambda r: (r, 0)),
        out_shape=jax.ShapeDtypeStruct((NP_, Hout), jnp.float32),
    )(z, sums, g.reshape(1, Hout), be.reshape(1, Hout))


def _pass2_head(z, sums, g, be, Lw1, Lb1, Lw2, Lb2):
    nb = 400

    def body(z_ref, sums_ref, g_ref, be_ref, w1_ref, b1_ref, w2_ref, b2_ref,
             out_ref):
        m = sums_ref[0:1, :] * (1.0 / N)
        var = sums_ref[1:2, :] * (1.0 / N) - m * m
        inv = lax.rsqrt(var + 1e-5)
        h = (z_ref[...] - m) * (inv * g_ref[...]) + be_ref[...]
        h = jnp.maximum(h, 0.0)
        t = jnp.dot(h, w1_ref[...], preferred_element_type=jnp.float32)
        t = jnp.maximum(t + b1_ref[...], 0.0)
        out_ref[...] = jnp.dot(
            t, w2_ref[...], preferred_element_type=jnp.float32) + b2_ref[...]

    return pl.pallas_call(
        body,
        grid=(N // nb,),
        in_specs=[
            pl.BlockSpec((nb, HE), lambda r: (r, 0)),
            pl.BlockSpec((2, HE), lambda r: (0, 0)),
            pl.BlockSpec((1, HE), lambda r: (0, 0)),
            pl.BlockSpec((1, HE), lambda r: (0, 0)),
            pl.BlockSpec((HE, H), lambda r: (0, 0)),
            pl.BlockSpec((1, H), lambda r: (0, 0)),
            pl.BlockSpec((H, OUT), lambda r: (0, 0)),
            pl.BlockSpec((1, OUT), lambda r: (0, 0)),
        ],
        out_specs=pl.BlockSpec((nb, OUT), lambda r: (r, 0)),
        out_shape=jax.ShapeDtypeStruct((N, OUT), jnp.float32),
    )(z, sums, g.reshape(1, HE), be.reshape(1, HE), Lw1,
      Lb1.reshape(1, H), Lw2, Lb2.reshape(1, OUT))


def kernel(x, edge_index, W1l, b1l, W1r, W2l, b2l, W2r, W3l, b3l, W3r,
           g1, be1, g2, be2, g3, be3, Lw1, Lb1, Lw2, Lb2):
    src = edge_index[0]
    dst = edge_index[1]
    pad = EP - E
    srcp = jnp.concatenate(
        [src, jnp.zeros((pad,), jnp.int32)]).reshape(ER, 128)
    pad_dst = N + jnp.arange(pad, dtype=jnp.int32) % (NP_ - N)
    dstp = jnp.concatenate([dst, pad_dst]).reshape(ER, 128)
    src_off = jnp.stack([srcp, srcp + NP_])
    src_same = jnp.stack([srcp, srcp])

    zrows = jnp.zeros((128, 128), jnp.float32)
    ones = jnp.ones((128, 128), jnp.float32)

    x2 = jnp.pad(x, ((0, NP_ - N), (0, 0)))
    x2d = jnp.concatenate([x2, x2], axis=0)

    deg2 = _sc_deg(dstp, zrows, ones)[:, :, :1]

    agg1 = _sc_agg(x2d, src_off, dstp, zrows, True)
    z1, s1 = _pass1(agg1, x2, deg2, W1l, b1l, W1r, D, H, True)
    h1 = _pass2(z1, s1, g1, be1, H)

    agg2 = _sc_agg(h1, src_same, dstp, zrows, False)
    z2, s2 = _pass1(agg2, h1, deg2, W2l, b2l, W2r, H, H, False)
    h2 = _pass2(z2, s2, g2, be2, H)

    agg3 = _sc_agg(h2, src_same, dstp, zrows, False)
    z3, s3 = _pass1(agg3, h2, deg2, W3l, b3l, W3r, H, HE, False)
    return _pass2_head(z3, s3, g3, be3, Lw1, Lb1, Lw2, Lb2)


# speedup vs baseline: 2.4579x; 2.2153x over previous
import jax
import jax.numpy as jnp
from jax import lax
from jax.experimental import pallas as pl
from jax.experimental.pallas import tpu as pltpu
from jax.experimental.pallas import tpu_sc as plsc

N = 10000
E = 320000
D = 128
H = 256
HE = 320
OUT = 64

NP_ = 10240
EP = 327680
ER = EP // 128
ZR = NP_ // 16
NBLK = 512
NB = NP_ // NBLK


def _sc_agg(h2, src_both, dstp, zrows, esplit):
    nrows = 80 if esplit else 160
    QR = 40
    NQ = nrows // QR
    mesh = plsc.VectorSubcoreMesh(core_axis_name="c", subcore_axis_name="s")
    scratch = [
        pltpu.VMEM_SHARED((NP_, 128), jnp.float32),
        pltpu.VMEM((QR, 128), jnp.int32),
        pltpu.VMEM((QR, 128), jnp.int32),
        pltpu.VMEM((2, 128, 128), jnp.float32),
        pltpu.SemaphoreType.DMA,
    ]

    def body(h_ref, srcb_ref, dst_ref, zr_ref, out_ref,
             acc, isrc, idst, rows, gsem):
        c = lax.axis_index("c")
        s = lax.axis_index("s")
        col = pl.multiple_of(c * 128, 128)

        def gather_src(r):
            if esplit:
                return h_ref.at[isrc.at[r]]
            return h_ref.at[isrc.at[r], pl.ds(col, 128)]

        pltpu.sync_copy(zr_ref, rows.at[0])
        for j in range(ZR // 128):
            pltpu.sync_copy(rows.at[0], acc.at[pl.ds(s * ZR + j * 128, 128)])
        plsc.subcore_barrier()

        base = (c * 16 + s) * nrows if esplit else s * nrows

        def quarter(q, carry):
            qbase = base + q * QR
            pltpu.sync_copy(srcb_ref.at[c, pl.ds(qbase, QR)], isrc)
            pltpu.sync_copy(dst_ref.at[pl.ds(qbase, QR)], idst)
            pltpu.async_copy(gather_src(0), rows.at[0], gsem)
            pltpu.async_copy(gather_src(1), rows.at[1], gsem)

            def step(r, cc):
                b = r % 2
                pltpu.make_async_copy(
                    h_ref.at[pl.ds(0, 128), pl.ds(0, 128)]
                    if not esplit else h_ref.at[pl.ds(0, 128)],
                    rows.at[b], gsem).wait()
                pltpu.sync_copy(rows.at[b], acc.at[idst.at[r]], add=True)

                @pl.when(r < QR - 2)
                def _():
                    pltpu.async_copy(gather_src(r + 2), rows.at[b], gsem)

                return cc

            lax.fori_loop(0, QR, step, 0)
            return carry

        lax.fori_loop(0, NQ, quarter, 0)
        plsc.subcore_barrier()

        for j in range(ZR // 128):
            pltpu.sync_copy(acc.at[pl.ds(s * ZR + j * 128, 128)], rows.at[0])
            pltpu.sync_copy(rows.at[0],
                            out_ref.at[c, pl.ds(s * ZR + j * 128, 128)])

    fn = pl.kernel(
        body,
        out_type=jax.ShapeDtypeStruct((2, NP_, 128), jnp.float32),
        mesh=mesh, scratch_types=scratch)
    return fn(h2, src_both, dstp, zrows)


def _sc_deg(dstp, zrows, ones):
    nrows = 80
    mesh = plsc.VectorSubcoreMesh(core_axis_name="c", subcore_axis_name="s")
    scratch = [
        pltpu.VMEM_SHARED((NP_, 128), jnp.float32),
        pltpu.VMEM((nrows, 128), jnp.int32),
        pltpu.VMEM((128, 128), jnp.float32),
    ]

    def body(dst_ref, zr_ref, ones_ref, out_ref, degacc, idst, ones_v):
        c = lax.axis_index("c")
        s = lax.axis_index("s")
        pltpu.sync_copy(zr_ref, ones_v)
        for j in range(ZR // 128):
            pltpu.sync_copy(ones_v,
                            degacc.at[pl.ds(s * ZR + j * 128, 128)])
        base = (c * 16 + s) * nrows
        pltpu.sync_copy(dst_ref.at[pl.ds(base, nrows)], idst)
        pltpu.sync_copy(ones_ref, ones_v)
        plsc.subcore_barrier()

        def step(r, cc):
            pltpu.sync_copy(ones_v, degacc.at[idst.at[r]], add=True)
            return cc

        lax.fori_loop(0, nrows, step, 0)
        plsc.subcore_barrier()
        for j in range(ZR // 128):
            pltpu.sync_copy(degacc.at[pl.ds(s * ZR + j * 128, 128)], ones_v)
            pltpu.sync_copy(ones_v,
                            out_ref.at[c, pl.ds(s * ZR + j * 128, 128)])

    fn = pl.kernel(
        body,
        out_type=jax.ShapeDtypeStruct((2, NP_, 128), jnp.float32),
        mesh=mesh, scratch_types=scratch)
    return fn(dstp, zrows, ones)


def _pass1(agg2, h2, deg2, Wl, bl, Wr, Din, Hout, esplit):
    aggr = agg2.reshape(2 * NP_, 128)

    def body(a0_ref, a1_ref, h_ref, d0_ref, d1_ref, wl_ref, bl_ref,
             wr_ref, z_ref, sums_ref):
        rb = pl.program_id(0)

        @pl.when(rb == 0)
        def _():
            sums_ref[...] = jnp.zeros_like(sums_ref)

        deg = d0_ref[0] + d1_ref[0]
        recip = 1.0 / jnp.maximum(deg, 1.0)
        hw = jnp.dot(h_ref[...], wr_ref[...],
                     preferred_element_type=jnp.float32)
        if esplit:
            a = (a0_ref[...] + a1_ref[...]) * recip
            z = hw + jnp.dot(a, wl_ref[...],
                             preferred_element_type=jnp.float32) + bl_ref[...]
        else:
            z = (hw
                 + jnp.dot(a0_ref[...] * recip, wl_ref[0:128, :],
                           preferred_element_type=jnp.float32)
                 + jnp.dot(a1_ref[...] * recip, wl_ref[128:, :],
                           preferred_element_type=jnp.float32)
                 + bl_ref[...])
        rid = rb * NBLK + lax.broadcasted_iota(jnp.int32, (NBLK, 1), 0)
        zm = jnp.where(rid < N, z, 0.0)
        sums_ref[0:1, :] += jnp.sum(zm, axis=0, keepdims=True)
        sums_ref[1:2, :] += jnp.sum(zm * zm, axis=0, keepdims=True)
        z_ref[...] = z

    return pl.pallas_call(
        body,
        grid=(NB,),
        in_specs=[
            pl.BlockSpec((NBLK, 128), lambda r: (r, 0)),
            pl.BlockSpec((NBLK, 128), lambda r: (r + NB, 0)),
            pl.BlockSpec((NBLK, Din), lambda r: (r, 0)),
            pl.BlockSpec((1, NBLK, 1), lambda r: (0, r, 0)),
            pl.BlockSpec((1, NBLK, 1), lambda r: (1, r, 0)),
            pl.BlockSpec((Din, Hout), lambda r: (0, 0)),
            pl.BlockSpec((1, Hout), lambda r: (0, 0)),
            pl.BlockSpec((Din, Hout), lambda r: (0, 0)),
        ],
        out_specs=[
            pl.BlockSpec((NBLK, Hout), lambda r: (r, 0)),
            pl.BlockSpec((2, Hout), lambda r: (0, 0)),
        ],
        out_shape=[
            jax.ShapeDtypeStruct((NP_, Hout), jnp.float32),
            jax.ShapeDtypeStruct((2, Hout), jnp.float32),
        ],
    )(aggr, aggr, h2, deg2, deg2, Wl, bl.reshape(1, Hout), Wr)


def _pass2(z, sums, g, be, Hout):
    def body(z_ref, sums_ref, g_ref, be_ref, out_ref):
        m = sums_ref[0:1, :] * (1.0 / N)
        var = sums_ref[1:2, :] * (1.0 / N) - m * m
        inv = lax.rsqrt(var + 1e-5)
        y = (z_ref[...] - m) * (inv * g_ref[...]) + be_ref[...]
        out_ref[...] = jnp.maximum(y, 0.0)

    return pl.pallas_call(
        body,
        grid=(NB,),
        in_specs=[
            pl.BlockSpec((NBLK, Hout), lambda r: (r, 0)),
            pl.BlockSpec((2, Hout), lambda r: (0, 0)),
            pl.BlockSpec((1, Hout), lambda r: (0, 0)),
            pl.BlockSpec((1, Hout), lambda r: (0, 0)),
        ],
        out_specs=pl.BlockSpec((NBLK, Hout), lambda r: (r, 0)),
        out_shape=jax.ShapeDtypeStruct((NP_, Hout), jnp.float32),
    )(z, sums, g.reshape(1, Hout), be.reshape(1, Hout))


def _pass2_head(z, sums, g, be, Lw1, Lb1, Lw2, Lb2):
    nb = 400

    def body(z_ref, sums_ref, g_ref, be_ref, w1_ref, b1_ref, w2_ref, b2_ref,
             out_ref):
        m = sums_ref[0:1, :] * (1.0 / N)
        var = sums_ref[1:2, :] * (1.0 / N) - m * m
        inv = lax.rsqrt(var + 1e-5)
        h = (z_ref[...] - m) * (inv * g_ref[...]) + be_ref[...]
        h = jnp.maximum(h, 0.0)
        t = jnp.dot(h, w1_ref[...], preferred_element_type=jnp.float32)
        t = jnp.maximum(t + b1_ref[...], 0.0)
        out_ref[...] = jnp.dot(
            t, w2_ref[...], preferred_element_type=jnp.float32) + b2_ref[...]

    return pl.pallas_call(
        body,
        grid=(N // nb,),
        in_specs=[
            pl.BlockSpec((nb, HE), lambda r: (r, 0)),
            pl.BlockSpec((2, HE), lambda r: (0, 0)),
            pl.BlockSpec((1, HE), lambda r: (0, 0)),
            pl.BlockSpec((1, HE), lambda r: (0, 0)),
            pl.BlockSpec((HE, H), lambda r: (0, 0)),
            pl.BlockSpec((1, H), lambda r: (0, 0)),
            pl.BlockSpec((H, OUT), lambda r: (0, 0)),
            pl.BlockSpec((1, OUT), lambda r: (0, 0)),
        ],
        out_specs=pl.BlockSpec((nb, OUT), lambda r: (r, 0)),
        out_shape=jax.ShapeDtypeStruct((N, OUT), jnp.float32),
    )(z, sums, g.reshape(1, HE), be.reshape(1, HE), Lw1,
      Lb1.reshape(1, H), Lw2, Lb2.reshape(1, OUT))


def kernel(x, edge_index, W1l, b1l, W1r, W2l, b2l, W2r, W3l, b3l, W3r,
           g1, be1, g2, be2, g3, be3, Lw1, Lb1, Lw2, Lb2):
    src = edge_index[0]
    dst = edge_index[1]
    pad = EP - E
    pad_src = jnp.arange(pad, dtype=jnp.int32) % N
    srcp = jnp.concatenate([src, pad_src]).reshape(ER, 128)
    pad_dst = N + jnp.arange(pad, dtype=jnp.int32) % (NP_ - N)
    dstp = jnp.concatenate([dst, pad_dst]).reshape(ER, 128)
    src_off = jnp.stack([srcp, srcp + NP_])
    src_same = jnp.stack([srcp, srcp])

    zrows = jnp.zeros((128, 128), jnp.float32)
    ones = jnp.ones((128, 128), jnp.float32)

    x2 = jnp.pad(x, ((0, NP_ - N), (0, 0)))
    x2d = jnp.concatenate([x2, x2], axis=0)

    deg2 = _sc_deg(dstp, zrows, ones)[:, :, :1]

    agg1 = _sc_agg(x2d, src_off, dstp, zrows, True)
    z1, s1 = _pass1(agg1, x2, deg2, W1l, b1l, W1r, D, H, True)
    h1 = _pass2(z1, s1, g1, be1, H)

    agg2 = _sc_agg(h1, src_same, dstp, zrows, False)
    z2, s2 = _pass1(agg2, h1, deg2, W2l, b2l, W2r, H, H, False)
    h2 = _pass2(z2, s2, g2, be2, H)

    agg3 = _sc_agg(h2, src_same, dstp, zrows, False)
    z3, s3 = _pass1(agg3, h2, deg2, W3l, b3l, W3r, H, HE, False)
    return _pass2_head(z3, s3, g3, be3, Lw1, Lb1, Lw2, Lb2)


# speedup vs baseline: 2.5742x; 1.0473x over previous
import jax
import jax.numpy as jnp
from jax import lax
from jax.experimental import pallas as pl
from jax.experimental.pallas import tpu as pltpu
from jax.experimental.pallas import tpu_sc as plsc

N = 10000
E = 320000
D = 128
H = 256
HE = 320
OUT = 64

NP_ = 10240
EP = 327680
ER = EP // 128
ZR = NP_ // 16
NBLK = 512
NB = NP_ // NBLK


def _sc_agg(h2, src_both, dstp, zrows, esplit):
    nrows = 80 if esplit else 160
    QR = 40
    NQ = nrows // QR
    mesh = plsc.VectorSubcoreMesh(core_axis_name="c", subcore_axis_name="s")
    scratch = [
        pltpu.VMEM_SHARED((NP_, 128), jnp.float32),
        pltpu.VMEM((QR, 128), jnp.int32),
        pltpu.VMEM((QR, 128), jnp.int32),
        pltpu.VMEM((2, 128, 128), jnp.float32),
        pltpu.SemaphoreType.DMA,
    ]

    def body(h_ref, srcb_ref, dst_ref, zr_ref, out_ref,
             acc, isrc, idst, rows, gsem):
        c = lax.axis_index("c")
        s = lax.axis_index("s")
        col = pl.multiple_of(c * 128, 128)

        def gather_src(r):
            if esplit:
                return h_ref.at[isrc.at[r]]
            return h_ref.at[isrc.at[r], pl.ds(col, 128)]

        pltpu.sync_copy(zr_ref, rows.at[0])
        for j in range(ZR // 128):
            pltpu.sync_copy(rows.at[0], acc.at[pl.ds(s * ZR + j * 128, 128)])
        plsc.subcore_barrier()

        base = (c * 16 + s) * nrows if esplit else s * nrows

        def quarter(q, carry):
            qbase = base + q * QR
            pltpu.sync_copy(srcb_ref.at[c, pl.ds(qbase, QR)], isrc)
            pltpu.sync_copy(dst_ref.at[pl.ds(qbase, QR)], idst)
            pltpu.async_copy(gather_src(0), rows.at[0], gsem)
            pltpu.async_copy(gather_src(1), rows.at[1], gsem)

            def step(r, cc):
                b = r % 2
                pltpu.make_async_copy(
                    h_ref.at[pl.ds(0, 128), pl.ds(0, 128)]
                    if not esplit else h_ref.at[pl.ds(0, 128)],
                    rows.at[b], gsem).wait()
                pltpu.sync_copy(rows.at[b], acc.at[idst.at[r]], add=True)

                @pl.when(r < QR - 2)
                def _():
                    pltpu.async_copy(gather_src(r + 2), rows.at[b], gsem)

                return cc

            lax.fori_loop(0, QR, step, 0)
            return carry

        lax.fori_loop(0, NQ, quarter, 0)
        plsc.subcore_barrier()

        for j in range(ZR // 128):
            pltpu.sync_copy(acc.at[pl.ds(s * ZR + j * 128, 128)], rows.at[0])
            pltpu.sync_copy(rows.at[0],
                            out_ref.at[c, pl.ds(s * ZR + j * 128, 128)])

    fn = pl.kernel(
        body,
        out_type=jax.ShapeDtypeStruct((2, NP_, 128), jnp.float32),
        mesh=mesh, scratch_types=scratch)
    return fn(h2, src_both, dstp, zrows)


def _sc_deg(dstp, zrows, ones):
    nrows = 80
    mesh = plsc.VectorSubcoreMesh(core_axis_name="c", subcore_axis_name="s")
    scratch = [
        pltpu.VMEM_SHARED((NP_, 128), jnp.float32),
        pltpu.VMEM((nrows, 128), jnp.int32),
        pltpu.VMEM((128, 128), jnp.float32),
    ]

    def body(dst_ref, zr_ref, ones_ref, out_ref, degacc, idst, ones_v):
        c = lax.axis_index("c")
        s = lax.axis_index("s")
        pltpu.sync_copy(zr_ref, ones_v)
        for j in range(ZR // 128):
            pltpu.sync_copy(ones_v,
                            degacc.at[pl.ds(s * ZR + j * 128, 128)])
        base = (c * 16 + s) * nrows
        pltpu.sync_copy(dst_ref.at[pl.ds(base, nrows)], idst)
        pltpu.sync_copy(ones_ref, ones_v)
        plsc.subcore_barrier()

        def step(r, cc):
            pltpu.sync_copy(ones_v, degacc.at[idst.at[r]], add=True)
            return cc

        lax.fori_loop(0, nrows, step, 0)
        plsc.subcore_barrier()
        for j in range(ZR // 128):
            pltpu.sync_copy(degacc.at[pl.ds(s * ZR + j * 128, 128)], ones_v)
            pltpu.sync_copy(ones_v,
                            out_ref.at[c, pl.ds(s * ZR + j * 128, 128)])

    fn = pl.kernel(
        body,
        out_type=jax.ShapeDtypeStruct((2, NP_, 128), jnp.float32),
        mesh=mesh, scratch_types=scratch)
    return fn(dstp, zrows, ones)


def _layer_tc(agg2, h2, deg2, Wl, bl, Wr, g, be, head, Din, Hout, esplit):
    aggr = agg2.reshape(2 * NP_, 128)
    Wout = OUT if head is not None else Hout

    def body(a0_ref, a1_ref, h_ref, d0_ref, d1_ref, wl_ref, bl_ref,
             wr_ref, g_ref, be_ref, *rest):
        if head is not None:
            w1_ref, b1_ref, w2_ref, b2_ref, out_ref, zbuf, sums = rest
        else:
            out_ref, zbuf, sums = rest
        p = pl.program_id(0)
        r = pl.program_id(1)

        @pl.when(jnp.logical_and(p == 0, r == 0))
        def _():
            sums[...] = jnp.zeros_like(sums)

        @pl.when(p == 0)
        def _():
            deg = d0_ref[0] + d1_ref[0]
            recip = 1.0 / jnp.maximum(deg, 1.0)
            hw = jnp.dot(h_ref[...], wr_ref[...],
                         preferred_element_type=jnp.float32)
            if esplit:
                a = (a0_ref[...] + a1_ref[...]) * recip
                z = hw + jnp.dot(
                    a, wl_ref[...],
                    preferred_element_type=jnp.float32) + bl_ref[...]
            else:
                z = (hw
                     + jnp.dot(a0_ref[...] * recip, wl_ref[0:128, :],
                               preferred_element_type=jnp.float32)
                     + jnp.dot(a1_ref[...] * recip, wl_ref[128:, :],
                               preferred_element_type=jnp.float32)
                     + bl_ref[...])
            rid = r * NBLK + lax.broadcasted_iota(jnp.int32, (NBLK, 1), 0)
            zm = jnp.where(rid < N, z, 0.0)
            sums[0:1, :] += jnp.sum(zm, axis=0, keepdims=True)
            sums[1:2, :] += jnp.sum(zm * zm, axis=0, keepdims=True)
            zbuf[pl.ds(r * NBLK, NBLK), :] = z

        @pl.when(p == 1)
        def _():
            z = zbuf[pl.ds(r * NBLK, NBLK), :]
            m = sums[0:1, :] * (1.0 / N)
            var = sums[1:2, :] * (1.0 / N) - m * m
            inv = lax.rsqrt(var + 1e-5)
            y = jnp.maximum((z - m) * (inv * g_ref[...]) + be_ref[...], 0.0)
            if head is not None:
                t = jnp.dot(y, w1_ref[...],
                            preferred_element_type=jnp.float32)
                t = jnp.maximum(t + b1_ref[...], 0.0)
                out_ref[...] = jnp.dot(
                    t, w2_ref[...],
                    preferred_element_type=jnp.float32) + b2_ref[...]
            else:
                out_ref[...] = y

    in_specs = [
        pl.BlockSpec((NBLK, 128), lambda p, r: (r * (1 - p), 0)),
        pl.BlockSpec((NBLK, 128), lambda p, r: ((r + NB) * (1 - p), 0)),
        pl.BlockSpec((NBLK, Din), lambda p, r: (r * (1 - p), 0)),
        pl.BlockSpec((1, NBLK, 1), lambda p, r: (0, r * (1 - p), 0)),
        pl.BlockSpec((1, NBLK, 1), lambda p, r: (1, r * (1 - p), 0)),
        pl.BlockSpec((Din, Hout), lambda p, r: (0, 0)),
        pl.BlockSpec((1, Hout), lambda p, r: (0, 0)),
        pl.BlockSpec((Din, Hout), lambda p, r: (0, 0)),
        pl.BlockSpec((1, Hout), lambda p, r: (0, 0)),
        pl.BlockSpec((1, Hout), lambda p, r: (0, 0)),
    ]
    args = [aggr, aggr, h2, deg2, deg2, Wl, bl.reshape(1, Hout), Wr,
            g.reshape(1, Hout), be.reshape(1, Hout)]
    if head is not None:
        Lw1, Lb1, Lw2, Lb2 = head
        in_specs += [
            pl.BlockSpec((HE, H), lambda p, r: (0, 0)),
            pl.BlockSpec((1, H), lambda p, r: (0, 0)),
            pl.BlockSpec((H, OUT), lambda p, r: (0, 0)),
            pl.BlockSpec((1, OUT), lambda p, r: (0, 0)),
        ]
        args += [Lw1, Lb1.reshape(1, H), Lw2, Lb2.reshape(1, OUT)]

    return pl.pallas_call(
        body,
        grid=(2, NB),
        in_specs=in_specs,
        out_specs=pl.BlockSpec((NBLK, Wout), lambda p, r: (r * p, 0)),
        out_shape=jax.ShapeDtypeStruct((NP_, Wout), jnp.float32),
        scratch_shapes=[
            pltpu.VMEM((NP_, Hout), jnp.float32),
            pltpu.VMEM((2, Hout), jnp.float32),
        ],
    )(*args)


def kernel(x, edge_index, W1l, b1l, W1r, W2l, b2l, W2r, W3l, b3l, W3r,
           g1, be1, g2, be2, g3, be3, Lw1, Lb1, Lw2, Lb2):
    src = edge_index[0]
    dst = edge_index[1]
    pad = EP - E
    pad_src = jnp.arange(pad, dtype=jnp.int32) % N
    srcp = jnp.concatenate([src, pad_src]).reshape(ER, 128)
    pad_dst = N + jnp.arange(pad, dtype=jnp.int32) % (NP_ - N)
    dstp = jnp.concatenate([dst, pad_dst]).reshape(ER, 128)
    src_off = jnp.stack([srcp, srcp + NP_])
    src_same = jnp.stack([srcp, srcp])

    zrows = jnp.zeros((128, 128), jnp.float32)
    ones = jnp.ones((128, 128), jnp.float32)

    x2 = jnp.pad(x, ((0, NP_ - N), (0, 0)))
    x2d = jnp.concatenate([x2, x2], axis=0)

    deg2 = _sc_deg(dstp, zrows, ones)[:, :, :1]

    agg1 = _sc_agg(x2d, src_off, dstp, zrows, True)
    h1 = _layer_tc(agg1, x2, deg2, W1l, b1l, W1r, g1, be1, None, D, H, True)

    agg2 = _sc_agg(h1, src_same, dstp, zrows, False)
    h2 = _layer_tc(agg2, h1, deg2, W2l, b2l, W2r, g2, be2, None, H, H, False)

    agg3 = _sc_agg(h2, src_same, dstp, zrows, False)
    out = _layer_tc(agg3, h2, deg2, W3l, b3l, W3r, g3, be3,
                    (Lw1, Lb1, Lw2, Lb2), H, HE, False)
    return out[:N]
